# baseline (speedup 1.0000x reference)
import jax
import jax.numpy as jnp
from jax.experimental import pallas as pl
from jax.experimental.pallas import tpu as pltpu

_B, _N, _F, _E = 1024, 500, 64, 16
_TN = 10
_STEPS = _N // _TN
_AROWS = 2 * _TN


def _fused_body(xt_ref, wt_ref, bt_ref, wvb_ref, p_ref, bv_ref,
                actt_ref, val_ref, acc_ref):
    i = pl.program_id(0)

    @pl.when(i == 0)
    def _init():
        acc_ref[...] = jnp.zeros((_E, _B), jnp.float32)

    wt = wt_ref[...]
    bt = bt_ref[...]
    acc = acc_ref[...]
    for n in range(_TN):
        y = jnp.dot(wt, xt_ref[n], preferred_element_type=jnp.float32)
        z = jnp.maximum(y + bt, 0.0)
        acc = acc + z * wvb_ref[n * _E:(n + 1) * _E, :]
    acc_ref[...] = acc

    @pl.when(i == _STEPS - 1)
    def _finish():
        val_ref[...] = (jnp.sum(acc_ref[...], axis=0, keepdims=True)
                        + bv_ref[0, 0])

    @pl.when(i == 0)
    def _actions():
        a0 = jax.nn.sigmoid(p_ref[0, 0])
        a1 = jax.nn.sigmoid(p_ref[0, 1]) * 0.5
        r = jax.lax.broadcasted_iota(jnp.int32, (2 * _N, _B), 0)
        actt_ref[...] = jnp.where(r % 2 == 0, a0, a1)


def kernel(node_features_gen, W_embed, b_embed, param, W_val, b_val):
    xt = jnp.transpose(node_features_gen, (1, 2, 0))
    wt = W_embed.T

    actt, val = pl.pallas_call(
        _fused_body,
        grid=(_STEPS,),
        in_specs=[
            pl.BlockSpec((_TN, _F, _B), lambda i: (i, 0, 0)),
            pl.BlockSpec((_E, _F), lambda i: (0, 0)),
            pl.BlockSpec((_E, 1), lambda i: (0, 0)),
            pl.BlockSpec((_TN * _E, 1), lambda i: (i, 0)),
            pl.BlockSpec((1, 2), lambda i: (0, 0)),
            pl.BlockSpec((1, 1), lambda i: (0, 0)),
        ],
        out_specs=[
            pl.BlockSpec((2 * _N, _B), lambda i: (0, 0)),
            pl.BlockSpec((1, _B), lambda i: (0, 0)),
        ],
        out_shape=[
            jax.ShapeDtypeStruct((2 * _N, _B), jnp.float32),
            jax.ShapeDtypeStruct((1, _B), jnp.float32),
        ],
        scratch_shapes=[pltpu.VMEM((_E, _B), jnp.float32)],
    )(xt, wt, b_embed.reshape(_E, 1), W_val,
      param.reshape(1, 2), b_val.reshape(1, 1))
    return actt.T, val.reshape(_B)


# speedup vs baseline: 1.3423x; 1.3423x over previous
import jax
import jax.numpy as jnp
from jax.experimental import pallas as pl
from jax.experimental.pallas import tpu as pltpu

_B, _N, _F, _E = 1024, 500, 64, 16
_TN = 25
_STEPS = _N // _TN
_AROWS = 2 * _TN


def _fused_body(xt_ref, wt_ref, bt_ref, wvb_ref, p_ref, bv_ref,
                actt_ref, val_ref, acc_ref):
    i = pl.program_id(0)

    @pl.when(i == 0)
    def _init():
        acc_ref[...] = jnp.zeros((_E, _B), jnp.float32)

    wt = wt_ref[...]
    bt = bt_ref[...]
    acc = acc_ref[...]
    for n in range(_TN):
        y = jnp.dot(wt, xt_ref[n], preferred_element_type=jnp.float32)
        z = jnp.maximum(y + bt, 0.0)
        acc = acc + z * wvb_ref[n * _E:(n + 1) * _E, :]
    acc_ref[...] = acc

    @pl.when(i == _STEPS - 1)
    def _finish():
        val_ref[...] = (jnp.sum(acc_ref[...], axis=0, keepdims=True)
                        + bv_ref[0, 0])

    @pl.when(i == 0)
    def _actions():
        a0 = jax.nn.sigmoid(p_ref[0, 0])
        a1 = jax.nn.sigmoid(p_ref[0, 1]) * 0.5
        r = jax.lax.broadcasted_iota(jnp.int32, (2 * _N, _B), 0)
        actt_ref[...] = jnp.where(r % 2 == 0, a0, a1)


def kernel(node_features_gen, W_embed, b_embed, param, W_val, b_val):
    xt = jnp.transpose(node_features_gen, (1, 2, 0))
    wt = W_embed.T

    actt, val = pl.pallas_call(
        _fused_body,
        grid=(_STEPS,),
        in_specs=[
            pl.BlockSpec((_TN, _F, _B), lambda i: (i, 0, 0)),
            pl.BlockSpec((_E, _F), lambda i: (0, 0)),
            pl.BlockSpec((_E, 1), lambda i: (0, 0)),
            pl.BlockSpec((_TN * _E, 1), lambda i: (i, 0)),
            pl.BlockSpec((1, 2), lambda i: (0, 0)),
            pl.BlockSpec((1, 1), lambda i: (0, 0)),
        ],
        out_specs=[
            pl.BlockSpec((2 * _N, _B), lambda i: (0, 0)),
            pl.BlockSpec((1, _B), lambda i: (0, 0)),
        ],
        out_shape=[
            jax.ShapeDtypeStruct((2 * _N, _B), jnp.float32),
            jax.ShapeDtypeStruct((1, _B), jnp.float32),
        ],
        scratch_shapes=[pltpu.VMEM((_E, _B), jnp.float32)],
    )(xt, wt, b_embed.reshape(_E, 1), W_val,
      param.reshape(1, 2), b_val.reshape(1, 1))
    return actt.T, val.reshape(_B)


# speedup vs baseline: 1.4345x; 1.0687x over previous
import jax
import jax.numpy as jnp
from jax.experimental import pallas as pl
from jax.experimental.pallas import tpu as pltpu

_B, _N, _F, _E = 1024, 500, 64, 16
_TN = 25
_STEPS = _N // _TN
_ROWS = _TN * _E
_K = 512


def _fused_body(xt_ref, wt_ref, b16_ref, wvp_ref, p_ref, bv_ref,
                actt_ref, val_ref, z_ref, bt_ref):
    i = pl.program_id(0)

    @pl.when(i == 0)
    def _init():
        z_ref[_ROWS:, :] = jnp.zeros((_K - _ROWS, _B), jnp.float32)
        a0 = jax.nn.sigmoid(p_ref[0, 0])
        a1 = jax.nn.sigmoid(p_ref[0, 1]) * 0.5
        r = jax.lax.broadcasted_iota(jnp.int32, (2 * _N, _B), 0)
        actt_ref[...] = jnp.where(r % 2 == 0, a0, a1)
        val_ref[...] = jnp.full((1, _B), bv_ref[0, 0], jnp.float32)
        e_idx = jax.lax.broadcasted_iota(jnp.int32, (_E, 1), 0)
        bt = jnp.zeros((_E, 1), jnp.float32)
        for e in range(_E):
            bt = jnp.where(e_idx == e, b16_ref[0, e], bt)
        bt_ref[...] = bt

    wt = wt_ref[...]
    bt = bt_ref[...]
    for n in range(_TN):
        y = jnp.dot(wt, xt_ref[n], preferred_element_type=jnp.float32)
        z_ref[n * _E:(n + 1) * _E, :] = jnp.maximum(y + bt, 0.0)
    part = jnp.dot(wvp_ref[0], z_ref[...],
                   preferred_element_type=jnp.float32)
    val_ref[...] += part


def kernel(node_features_gen, W_embed, b_embed, param, W_val, b_val):
    xt = jnp.transpose(node_features_gen, (1, 2, 0))
    wt = W_embed.T
    wvp = jnp.pad(W_val.reshape(_STEPS, _ROWS),
                  ((0, 0), (0, _K - _ROWS))).reshape(_STEPS, 1, _K)

    actt, val = pl.pallas_call(
        _fused_body,
        grid=(_STEPS,),
        in_specs=[
            pl.BlockSpec((_TN, _F, _B), lambda i: (i, 0, 0)),
            pl.BlockSpec((_E, _F), lambda i: (0, 0)),
            pl.BlockSpec((1, _E), lambda i: (0, 0)),
            pl.BlockSpec((1, 1, _K), lambda i: (i, 0, 0)),
            pl.BlockSpec((1, 2), lambda i: (0, 0)),
            pl.BlockSpec((1, 1), lambda i: (0, 0)),
        ],
        out_specs=[
            pl.BlockSpec((2 * _N, _B), lambda i: (0, 0)),
            pl.BlockSpec((1, _B), lambda i: (0, 0)),
        ],
        out_shape=[
            jax.ShapeDtypeStruct((2 * _N, _B), jnp.float32),
            jax.ShapeDtypeStruct((1, _B), jnp.float32),
        ],
        scratch_shapes=[pltpu.VMEM((_K, _B), jnp.float32),
                        pltpu.VMEM((_E, 1), jnp.float32)],
    )(xt, wt, b_embed.reshape(1, _E), wvp,
      param.reshape(1, 2), b_val.reshape(1, 1))
    return actt.T, val.reshape(_B)


# speedup vs baseline: 1.4465x; 1.0084x over previous
import jax
import jax.numpy as jnp
from jax.experimental import pallas as pl
from jax.experimental.pallas import tpu as pltpu

_B, _N, _F, _E = 1024, 500, 64, 16
_TN = 25
_STEPS = _N // _TN
_ROWS = _TN * _E


def _fused_body(xt_ref, wt_ref, b16_ref, wvp_ref, p_ref, bv_ref,
                actt_ref, val_ref, z_ref, bt_ref):
    i = pl.program_id(0)

    @pl.when(i == 0)
    def _init():
        a0 = jax.nn.sigmoid(p_ref[0, 0])
        a1 = jax.nn.sigmoid(p_ref[0, 1]) * 0.5
        r = jax.lax.broadcasted_iota(jnp.int32, (2 * _N, _B), 0)
        actt_ref[...] = jnp.where(r % 2 == 0, a0, a1)
        val_ref[...] = jnp.full((1, _B), bv_ref[0, 0], jnp.float32)
        e_idx = jax.lax.broadcasted_iota(jnp.int32, (_E, 1), 0)
        bt = jnp.zeros((_E, 1), jnp.float32)
        for e in range(_E):
            bt = jnp.where(e_idx == e, b16_ref[0, e], bt)
        bt_ref[...] = bt

    wt = wt_ref[...]
    bt = bt_ref[...]
    for n in range(_TN):
        y = jnp.dot(wt, xt_ref[n], preferred_element_type=jnp.float32)
        z_ref[n * _E:(n + 1) * _E, :] = jnp.maximum(y + bt, 0.0)
    part = jnp.dot(wvp_ref[0], z_ref[...],
                   preferred_element_type=jnp.float32)
    val_ref[...] += part


def kernel(node_features_gen, W_embed, b_embed, param, W_val, b_val):
    xt = jnp.transpose(node_features_gen, (1, 2, 0))
    wt = W_embed.T
    wvp = W_val.reshape(_STEPS, 1, _ROWS)

    actt, val = pl.pallas_call(
        _fused_body,
        grid=(_STEPS,),
        in_specs=[
            pl.BlockSpec((_TN, _F, _B), lambda i: (i, 0, 0)),
            pl.BlockSpec((_E, _F), lambda i: (0, 0)),
            pl.BlockSpec((1, _E), lambda i: (0, 0)),
            pl.BlockSpec((1, 1, _ROWS), lambda i: (i, 0, 0)),
            pl.BlockSpec((1, 2), lambda i: (0, 0)),
            pl.BlockSpec((1, 1), lambda i: (0, 0)),
        ],
        out_specs=[
            pl.BlockSpec((2 * _N, _B), lambda i: (0, 0)),
            pl.BlockSpec((1, _B), lambda i: (0, 0)),
        ],
        out_shape=[
            jax.ShapeDtypeStruct((2 * _N, _B), jnp.float32),
            jax.ShapeDtypeStruct((1, _B), jnp.float32),
        ],
        scratch_shapes=[pltpu.VMEM((_ROWS, _B), jnp.float32),
                        pltpu.VMEM((_E, 1), jnp.float32)],
    )(xt, wt, b_embed.reshape(1, _E), wvp,
      param.reshape(1, 2), b_val.reshape(1, 1))
    return actt.T, val.reshape(_B)
